# baseline (speedup 1.0000x reference)
import functools

import jax
import jax.numpy as jnp
from jax import lax
from jax.experimental import pallas as pl
from jax.experimental.pallas import tpu as pltpu
from jax.experimental.pallas import tpu_sc as plsc

B, S = 1024, 50
CONT_DIM = 13
N_CAT = 26
CARD = 100000
EMB = 16
N_ROWS = CONT_DIM + N_CAT

NC, NS = 2, 16
NW = NC * NS
F_PER_C = N_CAT // NC
CU = CONT_DIM * EMB
CU_ROUNDS = (CU + NW - 1) // NW
LANES = 16
D = 8


def _sc_body(cat_t, cont_t, emb_f, tab_t, out4,
             row_v, idx_v, val_v, cval_v, emb_v,
             sem_r, sem_i, sem_o):
    cid = lax.axis_index("c")
    sid = lax.axis_index("s")
    wid = sid * NC + cid
    pltpu.sync_copy(emb_f, emb_v)


    def row_copy(j, sem):
        return pltpu.make_async_copy(
            tab_t.at[cid * F_PER_C + j, sid, :], row_v, sem)

    row_copy(0, sem_r).start()

    def g_unit(j, carry):
        f = cid * F_PER_C + j
        e = sid

        row_copy(j, sem_r).wait()
        for q in range(D):
            pltpu.async_copy(cat_t.at[f, q, :], idx_v.at[q], sem_i.at[q])

        def s_body(s, c2):
            sl = lax.rem(s, D)
            pltpu.make_async_copy(cat_t.at[f, s, :], idx_v.at[sl],
                                  sem_i.at[sl]).wait()

            @pl.when(s >= D)
            def _():
                pltpu.make_async_copy(val_v.at[sl], out4.at[0, 0, 0, :],
                                      sem_o.at[sl]).wait()

            for i in range(B // LANES):
                sli = pl.ds(i * LANES, LANES)
                val_v[sl, sli] = plsc.load_gather(row_v, [idx_v[sl, sli]])

            pltpu.async_copy(val_v.at[sl], out4.at[s, CONT_DIM + f, e, :],
                             sem_o.at[sl])

            @pl.when(s + D < S)
            def _():
                pltpu.async_copy(cat_t.at[f, s + D, :], idx_v.at[sl],
                                 sem_i.at[sl])

            return c2

        lax.fori_loop(0, S, s_body, 0)

        @pl.when(j + 1 < F_PER_C)
        def _():
            row_copy(j + 1, sem_r).start()

        def g_drain(q, c2):
            pltpu.make_async_copy(val_v.at[q], out4.at[0, 0, 0, :],
                                  sem_o.at[q]).wait()
            return c2

        lax.fori_loop(0, D, g_drain, 0)

        @pl.when(j < CU_ROUNDS)
        def _():
            u = wid + NW * j

            @pl.when(u < CU)
            def _():
                c = u // EMB
                e2 = u % EMB
                scal = plsc.load_gather(
                    emb_v, [jnp.full((LANES,), c * EMB + e2,
                                     dtype=jnp.int32)])
                for q in range(D):
                    pltpu.async_copy(cont_t.at[c, q, :], cval_v.at[q],
                                     sem_i.at[q])

                def cs_body(s, c2):
                    sl = lax.rem(s, D)
                    pltpu.make_async_copy(cont_t.at[c, s, :],
                                          cval_v.at[sl], sem_i.at[sl]).wait()

                    @pl.when(s >= D)
                    def _():
                        pltpu.make_async_copy(val_v.at[sl],
                                              out4.at[0, 0, 0, :],
                                              sem_o.at[sl]).wait()

                    for i in range(B // LANES):
                        sli = pl.ds(i * LANES, LANES)
                        val_v[sl, sli] = cval_v[sl, sli] * scal

                    pltpu.async_copy(val_v.at[sl], out4.at[s, c, e2, :],
                                     sem_o.at[sl])

                    @pl.when(s + D < S)
                    def _():
                        pltpu.async_copy(cont_t.at[c, s + D, :],
                                         cval_v.at[sl], sem_i.at[sl])

                    return c2

                lax.fori_loop(0, S, cs_body, 0)

                def c_drain(q, c2):
                    pltpu.make_async_copy(val_v.at[q],
                                          out4.at[0, 0, 0, :],
                                          sem_o.at[q]).wait()
                    return c2

                lax.fori_loop(0, D, c_drain, 0)

        return carry

    lax.fori_loop(0, F_PER_C, g_unit, 0)


@jax.jit
def _sc_embed(cat_t, cont_t, emb_f, tab_t):
    mesh = plsc.VectorSubcoreMesh(
        core_axis_name="c", subcore_axis_name="s",
        num_cores=NC, num_subcores=NS,
    )
    run = pl.kernel(
        _sc_body,
        out_type=jax.ShapeDtypeStruct((S, N_ROWS, EMB, B), jnp.float32),
        mesh=mesh,
        scratch_types=[
            pltpu.VMEM((CARD,), jnp.float32),
            pltpu.VMEM((D, B), jnp.int32),
            pltpu.VMEM((D, B), jnp.float32),
            pltpu.VMEM((D, B), jnp.float32),
            pltpu.VMEM((CONT_DIM * EMB,), jnp.float32),
            pltpu.SemaphoreType.DMA,
            pltpu.SemaphoreType.DMA((D,)),
            pltpu.SemaphoreType.DMA((D,)),
        ],
        compiler_params=pltpu.CompilerParams(
            use_tc_tiling_on_sc=True, needs_layout_passes=False),
    )
    return run(cat_t, cont_t, emb_f, tab_t)


def kernel(continuous, categorical, cont_embedding, cat_tables):
    cat_t = jnp.transpose(categorical, (2, 1, 0))
    cont_t = jnp.transpose(continuous, (2, 1, 0))
    tab_t = jnp.transpose(cat_tables, (0, 2, 1))
    emb_f = cont_embedding.reshape(CONT_DIM * EMB)
    out4 = _sc_embed(cat_t, cont_t, emb_f, tab_t)
    return jnp.transpose(out4, (3, 0, 1, 2))


# speedup vs baseline: 1.0226x; 1.0226x over previous
import functools

import jax
import jax.numpy as jnp
from jax import lax
from jax.experimental import pallas as pl
from jax.experimental.pallas import tpu as pltpu
from jax.experimental.pallas import tpu_sc as plsc

B, S = 1024, 50
CONT_DIM = 13
N_CAT = 26
CARD = 100000
EMB = 16
N_ROWS = CONT_DIM + N_CAT

NC, NS = 2, 16
NW = NC * NS
GU_PER_W = (N_CAT * EMB) // NW
CU = CONT_DIM * EMB
CU_ROUNDS = (CU + NW - 1) // NW
LANES = 16
D = 8


def _sc_body(cat_t, cont_t, emb_f, tab_t, out4,
             row_v, idx_v, val_v, cval_v, emb_v, sem_r, sem_i, sem_o):
    wid = lax.axis_index("s") * NC + lax.axis_index("c")
    pltpu.sync_copy(emb_f, emb_v)

    def row_copy(j, sem):
        g = wid * GU_PER_W + j
        return pltpu.make_async_copy(
            tab_t.at[g // EMB, g % EMB, :], row_v, sem)

    row_copy(0, sem_r).start()

    def g_unit(j, carry):
        g = wid * GU_PER_W + j
        f = g // EMB
        e = g % EMB
        row_copy(j, sem_r).wait()
        for q in range(D):
            pltpu.async_copy(cat_t.at[f, q, :], idx_v.at[q], sem_i)

        def s_body(s, c2):
            sl = lax.rem(s, D)
            pltpu.make_async_copy(cat_t.at[f, s, :], idx_v.at[sl],
                                  sem_i).wait()

            @pl.when(s >= D)
            def _():
                pltpu.make_async_copy(val_v.at[sl], out4.at[0, 0, 0, :],
                                      sem_o).wait()

            for i in range(B // LANES):
                sli = pl.ds(i * LANES, LANES)
                val_v[sl, sli] = plsc.load_gather(row_v, [idx_v[sl, sli]])

            pltpu.async_copy(val_v.at[sl], out4.at[s, CONT_DIM + f, e, :],
                             sem_o)

            @pl.when(s + D < S)
            def _():
                pltpu.async_copy(cat_t.at[f, s + D, :], idx_v.at[sl], sem_i)

            return c2

        lax.fori_loop(0, S, s_body, 0)

        @pl.when(j + 1 < GU_PER_W)
        def _():
            row_copy(j + 1, sem_r).start()

        def g_drain(q, c2):
            pltpu.make_async_copy(val_v.at[q], out4.at[0, 0, 0, :],
                                  sem_o).wait()
            return c2

        lax.fori_loop(0, D, g_drain, 0)

        @pl.when(j < CU_ROUNDS)
        def _():
            u = wid + NW * j

            @pl.when(u < CU)
            def _():
                c = u // EMB
                e2 = u % EMB
                scal = plsc.load_gather(
                    emb_v, [jnp.full((LANES,), c * EMB + e2,
                                     dtype=jnp.int32)])
                for q in range(D):
                    pltpu.async_copy(cont_t.at[c, q, :], cval_v.at[q],
                                     sem_i)

                def cs_body(s, c2):
                    sl = lax.rem(s, D)
                    pltpu.make_async_copy(cont_t.at[c, s, :],
                                          cval_v.at[sl], sem_i).wait()

                    @pl.when(s >= D)
                    def _():
                        pltpu.make_async_copy(val_v.at[sl],
                                              out4.at[0, 0, 0, :],
                                              sem_o).wait()

                    for i in range(B // LANES):
                        sli = pl.ds(i * LANES, LANES)
                        val_v[sl, sli] = cval_v[sl, sli] * scal

                    pltpu.async_copy(val_v.at[sl], out4.at[s, c, e2, :],
                                     sem_o)

                    @pl.when(s + D < S)
                    def _():
                        pltpu.async_copy(cont_t.at[c, s + D, :],
                                         cval_v.at[sl], sem_i)

                    return c2

                lax.fori_loop(0, S, cs_body, 0)

                def c_drain(q, c2):
                    pltpu.make_async_copy(val_v.at[q],
                                          out4.at[0, 0, 0, :],
                                          sem_o).wait()
                    return c2

                lax.fori_loop(0, D, c_drain, 0)

        return carry

    lax.fori_loop(0, GU_PER_W, g_unit, 0)


@jax.jit
def _sc_embed(cat_t, cont_t, emb_f, tab_t):
    mesh = plsc.VectorSubcoreMesh(
        core_axis_name="c", subcore_axis_name="s",
        num_cores=NC, num_subcores=NS,
    )
    run = pl.kernel(
        _sc_body,
        out_type=jax.ShapeDtypeStruct((S, N_ROWS, EMB, B), jnp.float32),
        mesh=mesh,
        scratch_types=[
            pltpu.VMEM((CARD,), jnp.float32),
            pltpu.VMEM((D, B), jnp.int32),
            pltpu.VMEM((D, B), jnp.float32),
            pltpu.VMEM((D, B), jnp.float32),
            pltpu.VMEM((CONT_DIM * EMB,), jnp.float32),
            pltpu.SemaphoreType.DMA,
            pltpu.SemaphoreType.DMA,
            pltpu.SemaphoreType.DMA,
        ],
        compiler_params=pltpu.CompilerParams(
            use_tc_tiling_on_sc=True, needs_layout_passes=False),
    )
    return run(cat_t, cont_t, emb_f, tab_t)


def kernel(continuous, categorical, cont_embedding, cat_tables):
    cat_t = jnp.transpose(categorical, (2, 1, 0))
    cont_t = jnp.transpose(continuous, (2, 1, 0))
    tab_t = jnp.transpose(cat_tables, (0, 2, 1))
    emb_f = cont_embedding.reshape(CONT_DIM * EMB)
    out4 = _sc_embed(cat_t, cont_t, emb_f, tab_t)
    return jnp.transpose(out4, (3, 0, 1, 2))
